# baseline (speedup 1.0000x reference)
import functools

import jax
import jax.numpy as jnp
from jax import lax
from jax.experimental import pallas as pl
from jax.experimental.pallas import tpu as pltpu
from jax.experimental.pallas import tpu_sc as plsc

NC = 2
NS = 16
NW = NC * NS

N_NODES = 10000
N_ACC = 10240
ROWS_PER_TILE = N_ACC // NS


def _mesh():
    return plsc.VectorSubcoreMesh(
        core_axis_name="c", subcore_axis_name="s", num_cores=NC, num_subcores=NS
    )


DEG_ROWS = N_ACC // 128


@functools.partial(jax.jit, static_argnums=(1,))
def _sc_degree(dst2d, n_idx_rows_per_worker):
    rpw = n_idx_rows_per_worker
    npt = N_ACC // NS

    @functools.partial(
        pl.kernel,
        out_type=jax.ShapeDtypeStruct((NC, N_ACC), jnp.float32),
        mesh=_mesh(),
        compiler_params=pltpu.CompilerParams(needs_layout_passes=False),
        scratch_types=[
            pltpu.VMEM((rpw, 128), jnp.int32),
            pltpu.VMEM((N_ACC,), jnp.float32),
            pltpu.VMEM((NS, npt), jnp.float32),
            pltpu.VMEM_SHARED((NS, N_ACC), jnp.float32),
        ],
    )
    def deg_kernel(dst_hbm, out_hbm, dst_v, deg_loc, red_v, stage_sh):
        c = lax.axis_index("c")
        s = lax.axis_index("s")
        wid = c * NS + s
        zeros16 = jnp.zeros((16,), jnp.float32)
        ones16 = jnp.ones((16,), jnp.float32)

        def zero_body(i, carry):
            deg_loc[pl.ds(pl.multiple_of(i * 16, 16), 16)] = zeros16
            return carry

        lax.fori_loop(0, N_ACC // 16, zero_body, 0)
        pltpu.sync_copy(dst_hbm.at[pl.ds(wid * rpw, rpw)], dst_v)

        def body(i, carry):
            d16 = dst_v[i // 8, pl.ds(pl.multiple_of((i % 8) * 16, 16), 16)]
            plsc.addupdate_scatter(deg_loc, [d16], ones16)
            return carry

        lax.fori_loop(0, rpw * 8, body, 0)

        pltpu.sync_copy(deg_loc, stage_sh.at[s])
        plsc.subcore_barrier()
        pltpu.sync_copy(stage_sh.at[:, pl.ds(s * npt, npt)], red_v)

        def red_body(i, carry):
            off = pl.ds(pl.multiple_of(i * 16, 16), 16)
            acc = red_v[0, off]
            for k in range(1, NS):
                acc = acc + red_v[k, off]
            deg_loc[off] = acc
            return carry

        lax.fori_loop(0, npt // 16, red_body, 0)
        pltpu.sync_copy(deg_loc.at[pl.ds(0, npt)],
                        out_hbm.at[c, pl.ds(s * npt, npt)])

    return deg_kernel(dst2d)


@functools.partial(jax.jit, static_argnums=(3, 4))
def _sc_aggregate(h, src2d, dst2d, d_feat, n_idx_rows_per_worker):
    CR = 2
    n_outer = n_idx_rows_per_worker // CR
    zeros = jnp.zeros((ROWS_PER_TILE, d_feat), jnp.float32)

    @functools.partial(
        pl.kernel,
        out_type=jax.ShapeDtypeStruct((NC, N_ACC, d_feat), jnp.float32),
        mesh=_mesh(),
        scratch_types=[
            pltpu.VMEM((CR * 128,), jnp.int32),
            pltpu.VMEM((CR * 128,), jnp.int32),
            pltpu.VMEM((CR * 128, d_feat), jnp.float32),
            pltpu.VMEM_SHARED((N_ACC, d_feat), jnp.float32),
            pltpu.SemaphoreType.DMA,
        ],
    )
    def agg_kernel(h_hbm, src_hbm, dst_hbm, zeros_hbm, out_hbm,
                   src_v, dst_v, rows_v, agg_sh, gsem):
        c = lax.axis_index("c")
        s = lax.axis_index("s")
        wid = c * NS + s
        pltpu.sync_copy(zeros_hbm, agg_sh.at[pl.ds(s * ROWS_PER_TILE, ROWS_PER_TILE)])
        plsc.subcore_barrier()

        def body(i, carry):
            base = (wid * n_idx_rows_per_worker + i * CR) * 128
            pltpu.sync_copy(src_hbm.at[pl.ds(base, CR * 128)], src_v)
            pltpu.sync_copy(dst_hbm.at[pl.ds(base, CR * 128)], dst_v)
            pltpu.async_copy(h_hbm.at[src_v], rows_v, gsem).wait()
            pltpu.sync_copy(rows_v, agg_sh.at[dst_v], add=True)
            return carry

        lax.fori_loop(0, n_outer, body, 0)
        plsc.subcore_barrier()
        pltpu.sync_copy(
            agg_sh.at[pl.ds(s * ROWS_PER_TILE, ROWS_PER_TILE)],
            out_hbm.at[c, pl.ds(s * ROWS_PER_TILE, ROWS_PER_TILE)],
        )

    return agg_kernel(h, src2d, dst2d, zeros)


def _tc_first(deg_nm, x, w):
    n, d = x.shape

    def body(deg_ref, x_ref, w_ref, norm_ref, o_ref):
        dg = deg_ref[:, 0:1] + deg_ref[:, 1:2]
        nrm = jnp.where(dg > 0, 1.0 / jnp.sqrt(jnp.maximum(dg, 1.0)), 0.0)
        nrm2d = jnp.broadcast_to(nrm, (n, d))
        norm_ref[...] = nrm2d
        o_ref[...] = jnp.dot(x_ref[...], w_ref[...],
                             preferred_element_type=jnp.float32) * nrm2d

    return pl.pallas_call(
        body,
        out_shape=(
            jax.ShapeDtypeStruct((n, d), jnp.float32),
            jax.ShapeDtypeStruct((n, w.shape[1]), jnp.float32),
        ),
    )(deg_nm, x, w)


def _tc_post_and_next(parts, norm2d, b, w_next):
    n, d = norm2d.shape
    d_out = w_next.shape[1]

    def body(p_ref, norm_ref, b_ref, w_ref, o_ref):
        nrm = norm_ref[...]
        z = (p_ref[0, :n, :] + p_ref[1, :n, :]) * nrm + b_ref[...][None, :]
        z = jnp.maximum(z, 0.0)
        mu = jnp.mean(z)
        zc = z - mu
        var = jnp.mean(zc * zc)
        zn = zc / jnp.sqrt(var + 1e-5)
        o_ref[...] = jnp.dot(zn * nrm, w_ref[...],
                             preferred_element_type=jnp.float32)

    return pl.pallas_call(
        body,
        out_shape=jax.ShapeDtypeStruct((n, d_out), jnp.float32),
    )(parts, norm2d, b, w_next)


def _tc_final(parts, norm2d, b):
    n = norm2d.shape[0]
    d = b.shape[0]

    def body(p_ref, norm_ref, b_ref, o_ref):
        nrm = norm_ref[...][:, :d]
        o_ref[...] = (p_ref[0, :n, :d] + p_ref[1, :n, :d]) * nrm + b_ref[...][None, :]

    return pl.pallas_call(
        body,
        out_shape=jax.ShapeDtypeStruct((n, d), jnp.float32),
    )(parts, norm2d, b)


def kernel(features, edge_index, W0, b0, W1, b1, W2, b2, num_bits, num_grad_bits):
    n, _ = features.shape
    e = edge_index.shape[1]

    epw = ((e + NW - 1) // NW + 1023) // 1024 * 1024
    e_pad = epw * NW
    rows_per_worker = epw // 128
    pad_idx = jnp.arange(e_pad - e, dtype=jnp.int32)
    src = jnp.concatenate([edge_index[0], pad_idx % n])
    dst = jnp.concatenate([edge_index[1], n + pad_idx % (N_ACC - n)])
    dst2d = dst.reshape(e_pad // 128, 128)

    deg_parts = _sc_degree(dst2d, rows_per_worker)
    deg_nm = deg_parts[:, :n].T

    norm2d, m0 = _tc_first(deg_nm, features, W0)
    p0 = _sc_aggregate(m0, src, dst, 128, rows_per_worker)

    m1 = _tc_post_and_next(p0, norm2d, b0, W1)
    p1 = _sc_aggregate(m1, src, dst, 128, rows_per_worker)

    w2p = jnp.concatenate([W2, jnp.zeros((W2.shape[0], 128 - W2.shape[1]),
                                         jnp.float32)], axis=1)
    m2 = _tc_post_and_next(p1, norm2d, b1, w2p)
    p2 = _sc_aggregate(m2, src, dst, 128, rows_per_worker)

    return _tc_final(p2, norm2d, b2)


# speedup vs baseline: 1.3281x; 1.3281x over previous
import functools

import jax
import jax.numpy as jnp
from jax import lax
from jax.experimental import pallas as pl
from jax.experimental.pallas import tpu as pltpu
from jax.experimental.pallas import tpu_sc as plsc

NC = 2
NS = 16
NW = NC * NS

N_NODES = 10000
N_ACC = 10240
ROWS_PER_TILE = N_ACC // NS


def _mesh():
    return plsc.VectorSubcoreMesh(
        core_axis_name="c", subcore_axis_name="s", num_cores=NC, num_subcores=NS
    )


DEG_ROWS = N_ACC // 128


@functools.partial(jax.jit, static_argnums=(1,))
def _sc_degree(dst2d, n_idx_rows_per_worker):
    rpw = n_idx_rows_per_worker
    npt = N_ACC // NS

    @functools.partial(
        pl.kernel,
        out_type=jax.ShapeDtypeStruct((NC, N_ACC), jnp.float32),
        mesh=_mesh(),
        compiler_params=pltpu.CompilerParams(needs_layout_passes=False),
        scratch_types=[
            pltpu.VMEM((rpw, 128), jnp.int32),
            pltpu.VMEM((N_ACC,), jnp.float32),
            pltpu.VMEM((NS, npt), jnp.float32),
            pltpu.VMEM_SHARED((NS, N_ACC), jnp.float32),
        ],
    )
    def deg_kernel(dst_hbm, out_hbm, dst_v, deg_loc, red_v, stage_sh):
        c = lax.axis_index("c")
        s = lax.axis_index("s")
        wid = c * NS + s
        zeros16 = jnp.zeros((16,), jnp.float32)
        ones16 = jnp.ones((16,), jnp.float32)

        def zero_body(i, carry):
            deg_loc[pl.ds(pl.multiple_of(i * 16, 16), 16)] = zeros16
            return carry

        lax.fori_loop(0, N_ACC // 16, zero_body, 0)
        pltpu.sync_copy(dst_hbm.at[pl.ds(wid * rpw, rpw)], dst_v)

        def body(i, carry):
            d16 = dst_v[i // 8, pl.ds(pl.multiple_of((i % 8) * 16, 16), 16)]
            plsc.addupdate_scatter(deg_loc, [d16], ones16)
            return carry

        lax.fori_loop(0, rpw * 8, body, 0)

        pltpu.sync_copy(deg_loc, stage_sh.at[s])
        plsc.subcore_barrier()
        pltpu.sync_copy(stage_sh.at[:, pl.ds(s * npt, npt)], red_v)

        def red_body(i, carry):
            off = pl.ds(pl.multiple_of(i * 16, 16), 16)
            acc = red_v[0, off]
            for k in range(1, NS):
                acc = acc + red_v[k, off]
            deg_loc[off] = acc
            return carry

        lax.fori_loop(0, npt // 16, red_body, 0)
        pltpu.sync_copy(deg_loc.at[pl.ds(0, npt)],
                        out_hbm.at[c, pl.ds(s * npt, npt)])

    return deg_kernel(dst2d)


@functools.partial(jax.jit, static_argnums=(3, 4))
def _sc_aggregate(h, src2d, dst2d, d_feat, n_idx_rows_per_worker):
    G = 4
    NB = 2
    n_pairs = n_idx_rows_per_worker // G // 2
    zeros = jnp.zeros((ROWS_PER_TILE, d_feat), jnp.float32)

    @functools.partial(
        pl.kernel,
        out_type=jax.ShapeDtypeStruct((NC, N_ACC, d_feat), jnp.float32),
        mesh=_mesh(),
        scratch_types=[
            [pltpu.VMEM((G, 128), jnp.int32)] * 2,
            [pltpu.VMEM((G, 128), jnp.int32)] * 2,
            pltpu.VMEM((NB, 128, d_feat), jnp.float32),
            pltpu.VMEM_SHARED((N_ACC, d_feat), jnp.float32),
            [pltpu.SemaphoreType.DMA] * NB,
            [pltpu.SemaphoreType.DMA] * 2,
        ],
    )
    def agg_kernel(h_hbm, src_hbm, dst_hbm, zeros_hbm, out_hbm,
                   src_v, dst_v, rows_v, agg_sh, gsems, isems):
        c = lax.axis_index("c")
        s = lax.axis_index("s")
        wid = c * NS + s
        pltpu.sync_copy(zeros_hbm, agg_sh.at[pl.ds(s * ROWS_PER_TILE, ROWS_PER_TILE)])
        pltpu.sync_copy(src_hbm.at[pl.ds(wid * n_idx_rows_per_worker, G)],
                        src_v[0])
        pltpu.sync_copy(dst_hbm.at[pl.ds(wid * n_idx_rows_per_worker, G)],
                        dst_v[0])
        plsc.subcore_barrier()

        def run_group(p):
            gcps = [
                pltpu.async_copy(h_hbm.at[src_v[p].at[j]], rows_v.at[j],
                                 gsems[j])
                for j in range(NB)
            ]
            for j in range(G):
                b = j % NB
                gcps[b].wait()
                pltpu.sync_copy(rows_v.at[b], agg_sh.at[dst_v[p].at[j]],
                                add=True)
                if j + NB < G:
                    gcps[b] = pltpu.async_copy(
                        h_hbm.at[src_v[p].at[j + NB]], rows_v.at[b], gsems[b])

        def prefetch(p, base):
            return (
                pltpu.async_copy(src_hbm.at[pl.ds(base, G)], src_v[p],
                                 isems[p]),
                pltpu.async_copy(dst_hbm.at[pl.ds(base, G)], dst_v[p],
                                 isems[p]),
            )

        def body(i, carry):
            base = wid * n_idx_rows_per_worker + i * 2 * G
            cp_b = prefetch(1, base + G)
            run_group(0)
            cp_a = prefetch(0, base + 2 * G)
            for cp in cp_b:
                cp.wait()
            run_group(1)
            for cp in cp_a:
                cp.wait()
            return carry

        lax.fori_loop(0, n_pairs, body, 0)
        plsc.subcore_barrier()
        pltpu.sync_copy(
            agg_sh.at[pl.ds(s * ROWS_PER_TILE, ROWS_PER_TILE)],
            out_hbm.at[c, pl.ds(s * ROWS_PER_TILE, ROWS_PER_TILE)],
        )

    return agg_kernel(h, src2d, dst2d, zeros)


def _norm_col(deg_ref):
    dg = deg_ref[:, 0:1] + deg_ref[:, 1:2]
    return jnp.where(dg > 0, 1.0 / jnp.sqrt(jnp.maximum(dg, 1.0)), 0.0)


def _tc_first(deg_nm, x, w):
    n, d = x.shape

    def body(deg_ref, x_ref, w_ref, o_ref):
        o_ref[...] = jnp.dot(x_ref[...], w_ref[...],
                             preferred_element_type=jnp.float32) * _norm_col(deg_ref)

    return pl.pallas_call(
        body,
        out_shape=jax.ShapeDtypeStruct((n, w.shape[1]), jnp.float32),
    )(deg_nm, x, w)


def _tc_post_and_next(parts, deg_nm, b, w_next):
    n = deg_nm.shape[0]
    d_out = w_next.shape[1]

    def body(p_ref, deg_ref, b_ref, w_ref, o_ref):
        nrm = _norm_col(deg_ref)
        z = (p_ref[0, :n, :] + p_ref[1, :n, :]) * nrm + b_ref[...][None, :]
        z = jnp.maximum(z, 0.0)
        mu = jnp.mean(z)
        zc = z - mu
        var = jnp.mean(zc * zc)
        zn = zc / jnp.sqrt(var + 1e-5)
        o_ref[...] = jnp.dot(zn * nrm, w_ref[...],
                             preferred_element_type=jnp.float32)

    return pl.pallas_call(
        body,
        out_shape=jax.ShapeDtypeStruct((n, d_out), jnp.float32),
    )(parts, deg_nm, b, w_next)


def _tc_final(parts, deg_nm, b):
    n = deg_nm.shape[0]
    d = b.shape[0]

    def body(p_ref, deg_ref, b_ref, o_ref):
        nrm = _norm_col(deg_ref)
        o_ref[...] = (p_ref[0, :n, :d] + p_ref[1, :n, :d]) * nrm + b_ref[...][None, :]

    return pl.pallas_call(
        body,
        out_shape=jax.ShapeDtypeStruct((n, d), jnp.float32),
    )(parts, deg_nm, b)


def kernel(features, edge_index, W0, b0, W1, b1, W2, b2, num_bits, num_grad_bits):
    n, _ = features.shape
    e = edge_index.shape[1]

    epw = ((e + NW - 1) // NW + 1023) // 1024 * 1024
    e_pad = epw * NW
    rows_per_worker = epw // 128
    pad_idx = jnp.arange(e_pad + 512 - e, dtype=jnp.int32)
    src = jnp.concatenate(
        [edge_index[0], pad_idx % n]
    ).reshape(-1, 128)
    dst = jnp.concatenate(
        [edge_index[1], n + pad_idx % (N_ACC - n)]
    ).reshape(-1, 128)

    deg_parts = _sc_degree(dst, rows_per_worker)
    deg_nm = deg_parts[:, :n].T

    m0 = _tc_first(deg_nm, features, W0)
    p0 = _sc_aggregate(m0, src, dst, 128, rows_per_worker)

    m1 = _tc_post_and_next(p0, deg_nm, b0, W1)
    p1 = _sc_aggregate(m1, src, dst, 128, rows_per_worker)

    w2p = jnp.concatenate([W2, jnp.zeros((W2.shape[0], 128 - W2.shape[1]),
                                         jnp.float32)], axis=1)
    m2 = _tc_post_and_next(p1, deg_nm, b1, w2p)
    p2 = _sc_aggregate(m2, src, dst, 128, rows_per_worker)

    return _tc_final(p2, deg_nm, b2)


# speedup vs baseline: 1.3298x; 1.0013x over previous
import functools

import jax
import jax.numpy as jnp
from jax import lax
from jax.experimental import pallas as pl
from jax.experimental.pallas import tpu as pltpu
from jax.experimental.pallas import tpu_sc as plsc

NC = 2
NS = 16
NW = NC * NS

N_NODES = 10000
N_ACC = 10240
ROWS_PER_TILE = N_ACC // NS


def _mesh():
    return plsc.VectorSubcoreMesh(
        core_axis_name="c", subcore_axis_name="s", num_cores=NC, num_subcores=NS
    )


DEG_ROWS = N_ACC // 128


@functools.partial(jax.jit, static_argnums=(1,))
def _sc_degree(dst2d, n_idx_rows_per_worker):
    rpw = n_idx_rows_per_worker
    npt = N_ACC // NS

    @functools.partial(
        pl.kernel,
        out_type=jax.ShapeDtypeStruct((NC, N_ACC), jnp.float32),
        mesh=_mesh(),
        compiler_params=pltpu.CompilerParams(needs_layout_passes=False),
        scratch_types=[
            pltpu.VMEM((rpw, 128), jnp.int32),
            pltpu.VMEM((N_ACC,), jnp.float32),
            pltpu.VMEM((NS, npt), jnp.float32),
            pltpu.VMEM_SHARED((NS, N_ACC), jnp.float32),
        ],
    )
    def deg_kernel(dst_hbm, out_hbm, dst_v, deg_loc, red_v, stage_sh):
        c = lax.axis_index("c")
        s = lax.axis_index("s")
        wid = c * NS + s
        zeros16 = jnp.zeros((16,), jnp.float32)
        ones16 = jnp.ones((16,), jnp.float32)

        def zero_body(i, carry):
            deg_loc[pl.ds(pl.multiple_of(i * 16, 16), 16)] = zeros16
            return carry

        lax.fori_loop(0, N_ACC // 16, zero_body, 0)
        pltpu.sync_copy(dst_hbm.at[pl.ds(wid * rpw, rpw)], dst_v)

        def body(i, carry):
            d16 = dst_v[i // 8, pl.ds(pl.multiple_of((i % 8) * 16, 16), 16)]
            plsc.addupdate_scatter(deg_loc, [d16], ones16)
            return carry

        lax.fori_loop(0, rpw * 8, body, 0)

        pltpu.sync_copy(deg_loc, stage_sh.at[s])
        plsc.subcore_barrier()
        pltpu.sync_copy(stage_sh.at[:, pl.ds(s * npt, npt)], red_v)

        def red_body(i, carry):
            off = pl.ds(pl.multiple_of(i * 16, 16), 16)
            acc = red_v[0, off]
            for k in range(1, NS):
                acc = acc + red_v[k, off]
            deg_loc[off] = acc
            return carry

        lax.fori_loop(0, npt // 16, red_body, 0)
        pltpu.sync_copy(deg_loc.at[pl.ds(0, npt)],
                        out_hbm.at[c, pl.ds(s * npt, npt)])

    return deg_kernel(dst2d)


@functools.partial(jax.jit, static_argnums=(3, 4))
def _sc_aggregate(h, src2d, dst2d, d_feat, n_idx_rows_per_worker):
    G = 8
    NB = 2
    n_outer = n_idx_rows_per_worker // G
    zeros = jnp.zeros((ROWS_PER_TILE, d_feat), jnp.float32)

    @functools.partial(
        pl.kernel,
        out_type=jax.ShapeDtypeStruct((NC, N_ACC, d_feat), jnp.float32),
        mesh=_mesh(),
        scratch_types=[
            pltpu.VMEM((G, 128), jnp.int32),
            pltpu.VMEM((G, 128), jnp.int32),
            pltpu.VMEM((NB, 128, d_feat), jnp.float32),
            pltpu.VMEM_SHARED((N_ACC, d_feat), jnp.float32),
            [pltpu.SemaphoreType.DMA] * NB,
        ],
    )
    def agg_kernel(h_hbm, src_hbm, dst_hbm, zeros_hbm, out_hbm,
                   src_v, dst_v, rows_v, agg_sh, gsems):
        c = lax.axis_index("c")
        s = lax.axis_index("s")
        wid = c * NS + s
        pltpu.sync_copy(zeros_hbm, agg_sh.at[pl.ds(s * ROWS_PER_TILE, ROWS_PER_TILE)])
        plsc.subcore_barrier()

        def body(i, carry):
            base = wid * n_idx_rows_per_worker + i * G
            pltpu.sync_copy(src_hbm.at[pl.ds(base, G)], src_v)
            pltpu.sync_copy(dst_hbm.at[pl.ds(base, G)], dst_v)
            gcps = [
                pltpu.async_copy(h_hbm.at[src_v.at[j]], rows_v.at[j], gsems[j])
                for j in range(NB)
            ]
            for j in range(G):
                b = j % NB
                gcps[b].wait()
                pltpu.sync_copy(rows_v.at[b], agg_sh.at[dst_v.at[j]],
                                add=True)
                if j + NB < G:
                    gcps[b] = pltpu.async_copy(
                        h_hbm.at[src_v.at[j + NB]], rows_v.at[b], gsems[b])
            return carry

        lax.fori_loop(0, n_outer, body, 0)
        plsc.subcore_barrier()
        pltpu.sync_copy(
            agg_sh.at[pl.ds(s * ROWS_PER_TILE, ROWS_PER_TILE)],
            out_hbm.at[c, pl.ds(s * ROWS_PER_TILE, ROWS_PER_TILE)],
        )

    return agg_kernel(h, src2d, dst2d, zeros)


def _norm_col(deg_ref):
    dg = deg_ref[:, 0:1] + deg_ref[:, 1:2]
    return jnp.where(dg > 0, 1.0 / jnp.sqrt(jnp.maximum(dg, 1.0)), 0.0)


def _tc_first(deg_nm, x, w):
    n, d = x.shape

    def body(deg_ref, x_ref, w_ref, o_ref):
        o_ref[...] = jnp.dot(x_ref[...], w_ref[...],
                             preferred_element_type=jnp.float32) * _norm_col(deg_ref)

    return pl.pallas_call(
        body,
        out_shape=jax.ShapeDtypeStruct((n, w.shape[1]), jnp.float32),
    )(deg_nm, x, w)


def _tc_post_and_next(parts, deg_nm, b, w_next):
    n = deg_nm.shape[0]
    d_out = w_next.shape[1]

    def body(p_ref, deg_ref, b_ref, w_ref, o_ref):
        nrm = _norm_col(deg_ref)
        z = (p_ref[0, :n, :] + p_ref[1, :n, :]) * nrm + b_ref[...][None, :]
        z = jnp.maximum(z, 0.0)
        mu = jnp.mean(z)
        zc = z - mu
        var = jnp.mean(zc * zc)
        zn = zc / jnp.sqrt(var + 1e-5)
        o_ref[...] = jnp.dot(zn * nrm, w_ref[...],
                             preferred_element_type=jnp.float32)

    return pl.pallas_call(
        body,
        out_shape=jax.ShapeDtypeStruct((n, d_out), jnp.float32),
    )(parts, deg_nm, b, w_next)


def _tc_final(parts, deg_nm, b):
    n = deg_nm.shape[0]
    d = b.shape[0]

    def body(p_ref, deg_ref, b_ref, o_ref):
        nrm = _norm_col(deg_ref)
        o_ref[...] = (p_ref[0, :n, :d] + p_ref[1, :n, :d]) * nrm + b_ref[...][None, :]

    return pl.pallas_call(
        body,
        out_shape=jax.ShapeDtypeStruct((n, d), jnp.float32),
    )(parts, deg_nm, b)


def kernel(features, edge_index, W0, b0, W1, b1, W2, b2, num_bits, num_grad_bits):
    n, _ = features.shape
    e = edge_index.shape[1]

    epw = ((e + NW - 1) // NW + 1023) // 1024 * 1024
    e_pad = epw * NW
    rows_per_worker = epw // 128
    pad_idx = jnp.arange(e_pad + 512 - e, dtype=jnp.int32)
    src = jnp.concatenate(
        [edge_index[0], pad_idx % n]
    ).reshape(-1, 128)
    dst = jnp.concatenate(
        [edge_index[1], n + pad_idx % (N_ACC - n)]
    ).reshape(-1, 128)

    deg_parts = _sc_degree(dst, rows_per_worker)
    deg_nm = deg_parts[:, :n].T

    m0 = _tc_first(deg_nm, features, W0)
    p0 = _sc_aggregate(m0, src, dst, 128, rows_per_worker)

    m1 = _tc_post_and_next(p0, deg_nm, b0, W1)
    p1 = _sc_aggregate(m1, src, dst, 128, rows_per_worker)

    w2p = jnp.concatenate([W2, jnp.zeros((W2.shape[0], 128 - W2.shape[1]),
                                         jnp.float32)], axis=1)
    m2 = _tc_post_and_next(p1, deg_nm, b1, w2p)
    p2 = _sc_aggregate(m2, src, dst, 128, rows_per_worker)

    return _tc_final(p2, deg_nm, b2)


# speedup vs baseline: 1.3430x; 1.0099x over previous
import functools

import jax
import jax.numpy as jnp
from jax import lax
from jax.experimental import pallas as pl
from jax.experimental.pallas import tpu as pltpu
from jax.experimental.pallas import tpu_sc as plsc

NC = 2
NS = 16
NW = NC * NS

N_NODES = 10000
N_ACC = 10240
ROWS_PER_TILE = N_ACC // NS


def _mesh():
    return plsc.VectorSubcoreMesh(
        core_axis_name="c", subcore_axis_name="s", num_cores=NC, num_subcores=NS
    )


DEG_ROWS = N_ACC // 128


@functools.partial(jax.jit, static_argnums=(1,))
def _sc_degree(dst2d, n_idx_rows_per_worker):
    rpw = n_idx_rows_per_worker
    npt = N_ACC // NS

    @functools.partial(
        pl.kernel,
        out_type=jax.ShapeDtypeStruct((NC, N_ACC), jnp.float32),
        mesh=_mesh(),
        compiler_params=pltpu.CompilerParams(needs_layout_passes=False),
        scratch_types=[
            pltpu.VMEM((rpw, 128), jnp.int32),
            pltpu.VMEM((N_ACC,), jnp.float32),
            pltpu.VMEM((NS, npt), jnp.float32),
            pltpu.VMEM_SHARED((NS, N_ACC), jnp.float32),
        ],
    )
    def deg_kernel(dst_hbm, out_hbm, dst_v, deg_loc, red_v, stage_sh):
        c = lax.axis_index("c")
        s = lax.axis_index("s")
        wid = c * NS + s
        zeros16 = jnp.zeros((16,), jnp.float32)
        ones16 = jnp.ones((16,), jnp.float32)

        def zero_body(i, carry):
            deg_loc[pl.ds(pl.multiple_of(i * 16, 16), 16)] = zeros16
            return carry

        lax.fori_loop(0, N_ACC // 16, zero_body, 0)
        pltpu.sync_copy(dst_hbm.at[pl.ds(wid * rpw, rpw)], dst_v)

        def body(i, carry):
            d16 = dst_v[i // 8, pl.ds(pl.multiple_of((i % 8) * 16, 16), 16)]
            plsc.addupdate_scatter(deg_loc, [d16], ones16)
            return carry

        lax.fori_loop(0, rpw * 8, body, 0)

        pltpu.sync_copy(deg_loc, stage_sh.at[s])
        plsc.subcore_barrier()
        pltpu.sync_copy(stage_sh.at[:, pl.ds(s * npt, npt)], red_v)

        def red_body(i, carry):
            off = pl.ds(pl.multiple_of(i * 16, 16), 16)
            acc = red_v[0, off]
            for k in range(1, NS):
                acc = acc + red_v[k, off]
            deg_loc[off] = acc
            return carry

        lax.fori_loop(0, npt // 16, red_body, 0)
        pltpu.sync_copy(deg_loc.at[pl.ds(0, npt)],
                        out_hbm.at[c, pl.ds(s * npt, npt)])

    return deg_kernel(dst2d)


@functools.partial(jax.jit, static_argnums=(3, 4))
def _sc_aggregate(h, src2d, dst2d, d_feat, n_idx_rows_per_worker):
    G = 8
    NB = 2
    n_outer = n_idx_rows_per_worker // G
    zeros = jnp.zeros((ROWS_PER_TILE, d_feat), jnp.float32)

    @functools.partial(
        pl.kernel,
        out_type=jax.ShapeDtypeStruct((NC, N_ACC, d_feat), jnp.float32),
        mesh=_mesh(),
        scratch_types=[
            pltpu.VMEM((G, 128), jnp.int32),
            pltpu.VMEM((G, 128), jnp.int32),
            pltpu.VMEM((NB, 128, d_feat), jnp.float32),
            pltpu.VMEM_SHARED((N_ACC, d_feat), jnp.float32),
            [pltpu.SemaphoreType.DMA] * NB,
            [pltpu.SemaphoreType.DMA] * NB,
        ],
    )
    def agg_kernel(h_hbm, src_hbm, dst_hbm, zeros_hbm, out_hbm,
                   src_v, dst_v, rows_v, agg_sh, gsems, ssems):
        c = lax.axis_index("c")
        s = lax.axis_index("s")
        wid = c * NS + s
        pltpu.sync_copy(zeros_hbm, agg_sh.at[pl.ds(s * ROWS_PER_TILE, ROWS_PER_TILE)])
        plsc.subcore_barrier()

        def body(i, carry):
            base = wid * n_idx_rows_per_worker + i * G
            pltpu.sync_copy(src_hbm.at[pl.ds(base, G)], src_v)
            pltpu.sync_copy(dst_hbm.at[pl.ds(base, G)], dst_v)
            gcps = [
                pltpu.async_copy(h_hbm.at[src_v.at[j]], rows_v.at[j], gsems[j])
                for j in range(NB)
            ]
            for j in range(G):
                b = j % NB
                gcps[b].wait()
                pltpu.sync_copy(rows_v.at[b], agg_sh.at[dst_v.at[j]],
                                add=True)
                if j + NB < G:
                    gcps[b] = pltpu.async_copy(
                        h_hbm.at[src_v.at[j + NB]], rows_v.at[b], gsems[b])
            return carry

        lax.fori_loop(0, n_outer, body, 0)
        plsc.subcore_barrier()
        pltpu.sync_copy(
            agg_sh.at[pl.ds(s * ROWS_PER_TILE, ROWS_PER_TILE)],
            out_hbm.at[c, pl.ds(s * ROWS_PER_TILE, ROWS_PER_TILE)],
        )

    return agg_kernel(h, src2d, dst2d, zeros)


def _tc_first(deg_nm, x, w):
    n, d = x.shape

    def body(deg_ref, x_ref, w_ref, norm_ref, o_ref):
        dg = deg_ref[:, 0:1] + deg_ref[:, 1:2]
        nrm = jnp.where(dg > 0, 1.0 / jnp.sqrt(jnp.maximum(dg, 1.0)), 0.0)
        nrm2d = jnp.broadcast_to(nrm, (n, d))
        norm_ref[...] = nrm2d
        o_ref[...] = jnp.dot(x_ref[...], w_ref[...],
                             preferred_element_type=jnp.float32) * nrm2d

    return pl.pallas_call(
        body,
        out_shape=(
            jax.ShapeDtypeStruct((n, d), jnp.float32),
            jax.ShapeDtypeStruct((n, w.shape[1]), jnp.float32),
        ),
    )(deg_nm, x, w)


def _tc_post_and_next(parts, norm2d, b, w_next):
    n, d = norm2d.shape
    d_out = w_next.shape[1]

    def body(p_ref, norm_ref, b_ref, w_ref, o_ref):
        nrm = norm_ref[...]
        z = (p_ref[0, :n, :] + p_ref[1, :n, :]) * nrm + b_ref[...][None, :]
        z = jnp.maximum(z, 0.0)
        mu = jnp.mean(z)
        zc = z - mu
        var = jnp.mean(zc * zc)
        zn = zc / jnp.sqrt(var + 1e-5)
        o_ref[...] = jnp.dot(zn * nrm, w_ref[...],
                             preferred_element_type=jnp.float32)

    return pl.pallas_call(
        body,
        out_shape=jax.ShapeDtypeStruct((n, d_out), jnp.float32),
    )(parts, norm2d, b, w_next)


def _tc_final(parts, norm2d, b):
    n = norm2d.shape[0]
    d = b.shape[0]

    def body(p_ref, norm_ref, b_ref, o_ref):
        nrm = norm_ref[...][:, :d]
        o_ref[...] = (p_ref[0, :n, :d] + p_ref[1, :n, :d]) * nrm + b_ref[...][None, :]

    return pl.pallas_call(
        body,
        out_shape=jax.ShapeDtypeStruct((n, d), jnp.float32),
    )(parts, norm2d, b)


def kernel(features, edge_index, W0, b0, W1, b1, W2, b2, num_bits, num_grad_bits):
    n, _ = features.shape
    e = edge_index.shape[1]

    epw = ((e + NW - 1) // NW + 1023) // 1024 * 1024
    e_pad = epw * NW
    rows_per_worker = epw // 128
    pad_idx = jnp.arange(e_pad - e, dtype=jnp.int32)
    src = jnp.concatenate(
        [edge_index[0], pad_idx % n]
    ).reshape(e_pad // 128, 128)
    dst = jnp.concatenate(
        [edge_index[1], n + pad_idx % (N_ACC - n)]
    ).reshape(e_pad // 128, 128)

    deg_parts = _sc_degree(dst, rows_per_worker)
    deg_nm = deg_parts[:, :n].T

    norm2d, m0 = _tc_first(deg_nm, features, W0)
    p0 = _sc_aggregate(m0, src, dst, 128, rows_per_worker)

    m1 = _tc_post_and_next(p0, norm2d, b0, W1)
    p1 = _sc_aggregate(m1, src, dst, 128, rows_per_worker)

    w2p = jnp.concatenate([W2, jnp.zeros((W2.shape[0], 128 - W2.shape[1]),
                                         jnp.float32)], axis=1)
    m2 = _tc_post_and_next(p1, norm2d, b1, w2p)
    p2 = _sc_aggregate(m2, src, dst, 128, rows_per_worker)

    return _tc_final(p2, norm2d, b2)
